# baseline (speedup 1.0000x reference)
import functools

import jax
import jax.numpy as jnp
from jax import lax
from jax.experimental import pallas as pl
from jax.experimental.pallas import tpu as pltpu
from jax.experimental.pallas import tpu_sc as plsc

EMBED = 32
NCODE = 1024
NTOK = 16384
BETA = 0.25

BLK = 4096
NBLK = NTOK // BLK

_LOSS_SCALE = (1.0 + BETA) / (NTOK * EMBED)


def _dist_body(z_ref, wt2_ref, z2_ref, digits_ref, idx_ref, loss_ref):
    zb = z_ref[...]
    wt2 = wt2_ref[...]
    z2 = z2_ref[...]
    w2 = jnp.sum(wt2 * wt2, axis=0, keepdims=True) * 0.25
    cross2 = jax.lax.dot_general(
        zb, wt2, (((1,), (0,)), ((), ())),
        preferred_element_type=jnp.float32,
        precision=jax.lax.Precision.DEFAULT)
    d = (z2 + w2) - cross2
    dmin = jnp.min(d, axis=1, keepdims=True)
    oh = jnp.where(d == dmin, 1.0, 0.0)
    prod = jax.lax.dot_general(
        oh, digits_ref[...], (((1,), (0,)), ((), ())),
        preferred_element_type=jnp.float32,
        precision=jax.lax.Precision.DEFAULT)
    fast = prod[:, 0:1] * 4.0 + prod[:, 1:2]
    idx_ref[0, 0, :] = fast[:, 0].astype(jnp.int32)

    @pl.when(jnp.any(prod[:, 2] > 1.5))
    def _():
        codes = jax.lax.broadcasted_iota(jnp.int32, d.shape, 1)
        idx_ref[0, 0, :] = jnp.min(jnp.where(d == dmin, codes, NCODE), axis=1)

    @pl.when(pl.program_id(0) == 0)
    def _():
        loss_ref[...] = jnp.zeros_like(loss_ref)

    loss_ref[...] += jnp.sum(dmin).reshape(1, 1) * _LOSS_SCALE


def _digit_cols():
    import numpy as np
    n = np.arange(NCODE)
    m = np.zeros((NCODE, 128), np.float32)
    m[:, 0] = n >> 2
    m[:, 1] = n & 3
    m[:, 2] = 1.0
    return jnp.asarray(m)


_dist_call = pl.pallas_call(
    _dist_body,
    grid=(NBLK,),
    in_specs=[
        pl.BlockSpec((BLK, EMBED), lambda i: (i, 0)),
        pl.BlockSpec((EMBED, NCODE), lambda i: (0, 0)),
        pl.BlockSpec((BLK, 1), lambda i: (i, 0)),
        pl.BlockSpec((NCODE, 128), lambda i: (0, 0)),
    ],
    out_specs=[
        pl.BlockSpec((1, 1, BLK), lambda i: (i, 0, 0)),
        pl.BlockSpec((1, 1), lambda i: (0, 0)),
    ],
    out_shape=[
        jax.ShapeDtypeStruct((NBLK, 1, BLK), jnp.int32),
        jax.ShapeDtypeStruct((1, 1), jnp.float32),
    ],
)


@functools.cache
def _make_sc_gather():
    info = plsc.get_sparse_core_info()
    ncores = info.num_cores
    nw = ncores * info.num_subcores
    bpw = NTOK // nw

    @functools.partial(
        pl.kernel,
        mesh=plsc.VectorSubcoreMesh(core_axis_name="c", subcore_axis_name="s"),
        compiler_params=pltpu.CompilerParams(use_tc_tiling_on_sc=False),
        out_type=jax.ShapeDtypeStruct((NTOK, EMBED), jnp.float32),
        scratch_types=[
            pltpu.VMEM((bpw,), jnp.int32),
            pltpu.VMEM((bpw, EMBED), jnp.float32),
            pltpu.SemaphoreType.DMA,
        ],
    )
    def sc_gather(table_hbm, idx_hbm, out_hbm, idx_v, rows_v, sem):
        wid = lax.axis_index("s") * ncores + lax.axis_index("c")
        base = wid * bpw
        pltpu.sync_copy(idx_hbm.at[pl.ds(base, bpw)], idx_v)
        pltpu.async_copy(table_hbm.at[idx_v], rows_v, sem).wait()
        pltpu.sync_copy(rows_v, out_hbm.at[pl.ds(base, bpw)])

    return sc_gather


def kernel(z, W):
    z_flat = z.reshape(NTOK, EMBED)
    z2 = jnp.sum(z_flat ** 2, axis=1, keepdims=True)
    idx3, loss = _dist_call(z_flat, (2.0 * W).T, z2, _digit_cols())
    idx_flat = idx3.reshape(NTOK)
    z_q = _make_sc_gather()(W, idx_flat).reshape(z.shape)
    encoding_indices = idx_flat.reshape(z.shape[:-1])
    return (z_q, loss[0, 0], encoding_indices)


# speedup vs baseline: 1.0100x; 1.0100x over previous
import functools

import jax
import jax.numpy as jnp
from jax import lax
from jax.experimental import pallas as pl
from jax.experimental.pallas import tpu as pltpu
from jax.experimental.pallas import tpu_sc as plsc

EMBED = 32
NCODE = 1024
NTOK = 16384
BETA = 0.25

BLK = 4096
NBLK = NTOK // BLK

_LOSS_SCALE = (1.0 + BETA) / (NTOK * EMBED)


def _dist_body(z_ref, w_ref, z2_ref, idx_ref, loss_ref):
    zb = z_ref[...]
    wt2 = jnp.transpose(w_ref[...]) * 2.0
    z2 = z2_ref[...]
    w2 = jnp.sum(wt2 * wt2, axis=0, keepdims=True) * 0.25
    cross2 = jax.lax.dot_general(
        zb, wt2, (((1,), (0,)), ((), ())),
        preferred_element_type=jnp.float32,
        precision=jax.lax.Precision.DEFAULT)
    d = (z2 + w2) - cross2
    dmin = jnp.min(d, axis=1, keepdims=True)
    codes = jax.lax.broadcasted_iota(jnp.int32, d.shape, 1)
    idx_ref[0, 0, :] = jnp.min(jnp.where(d == dmin, codes, NCODE), axis=1)

    @pl.when(pl.program_id(0) == 0)
    def _():
        loss_ref[...] = jnp.zeros_like(loss_ref)

    loss_ref[...] += jnp.sum(dmin).reshape(1, 1) * _LOSS_SCALE


_dist_call = pl.pallas_call(
    _dist_body,
    grid=(NBLK,),
    in_specs=[
        pl.BlockSpec((BLK, EMBED), lambda i: (i, 0)),
        pl.BlockSpec((NCODE, EMBED), lambda i: (0, 0)),
        pl.BlockSpec((BLK, 1), lambda i: (i, 0)),
    ],
    out_specs=[
        pl.BlockSpec((1, 1, BLK), lambda i: (i, 0, 0)),
        pl.BlockSpec((1, 1), lambda i: (0, 0)),
    ],
    out_shape=[
        jax.ShapeDtypeStruct((NBLK, 1, BLK), jnp.int32),
        jax.ShapeDtypeStruct((1, 1), jnp.float32),
    ],
)


@functools.cache
def _make_sc_gather():
    info = plsc.get_sparse_core_info()
    ncores = info.num_cores
    nw = ncores * info.num_subcores
    bpw = NTOK // nw

    @functools.partial(
        pl.kernel,
        mesh=plsc.VectorSubcoreMesh(core_axis_name="c", subcore_axis_name="s"),
        compiler_params=pltpu.CompilerParams(use_tc_tiling_on_sc=False),
        out_type=jax.ShapeDtypeStruct((NTOK, EMBED), jnp.float32),
        scratch_types=[
            pltpu.VMEM((bpw,), jnp.int32),
            pltpu.VMEM((bpw, EMBED), jnp.float32),
            pltpu.SemaphoreType.DMA,
        ],
    )
    def sc_gather(table_hbm, idx_hbm, out_hbm, idx_v, rows_v, sem):
        wid = lax.axis_index("s") * ncores + lax.axis_index("c")
        base = wid * bpw
        pltpu.sync_copy(idx_hbm.at[pl.ds(base, bpw)], idx_v)
        pltpu.async_copy(table_hbm.at[idx_v], rows_v, sem).wait()
        pltpu.sync_copy(rows_v, out_hbm.at[pl.ds(base, bpw)])

    return sc_gather


def kernel(z, W):
    z_flat = z.reshape(NTOK, EMBED)
    z2 = jnp.sum(z_flat ** 2, axis=1, keepdims=True)
    idx3, loss = _dist_call(z_flat, W, z2)
    idx_flat = idx3.reshape(NTOK)
    z_q = _make_sc_gather()(W, idx_flat).reshape(z.shape)
    encoding_indices = idx_flat.reshape(z.shape[:-1])
    return (z_q, loss[0, 0], encoding_indices)


# speedup vs baseline: 1.0270x; 1.0168x over previous
import functools

import jax
import jax.numpy as jnp
from jax import lax
from jax.experimental import pallas as pl
from jax.experimental.pallas import tpu as pltpu
from jax.experimental.pallas import tpu_sc as plsc

EMBED = 32
NCODE = 1024
NTOK = 16384
BETA = 0.25

BLK = 4096
NBLK = NTOK // BLK

_LOSS_SCALE = (1.0 + BETA) / (NTOK * EMBED)


def _dist_body(z_ref, wt2_ref, z2_ref, idx_ref, loss_ref):
    zb = z_ref[...]
    wt2 = wt2_ref[...]
    z2 = z2_ref[...]
    w2 = jnp.sum(wt2 * wt2, axis=0, keepdims=True) * 0.25
    cross2 = jax.lax.dot_general(
        zb, wt2, (((1,), (0,)), ((), ())),
        preferred_element_type=jnp.float32,
        precision=jax.lax.Precision.DEFAULT)
    d = (z2 + w2) - cross2
    dmin = jnp.min(d, axis=1, keepdims=True)
    codes = jax.lax.broadcasted_iota(jnp.int32, d.shape, 1)
    idx_ref[0, 0, :] = jnp.min(jnp.where(d == dmin, codes, NCODE), axis=1)

    @pl.when(pl.program_id(0) == 0)
    def _():
        loss_ref[...] = jnp.zeros_like(loss_ref)

    loss_ref[...] += jnp.sum(dmin).reshape(1, 1) * _LOSS_SCALE


_dist_call = pl.pallas_call(
    _dist_body,
    grid=(NBLK,),
    in_specs=[
        pl.BlockSpec((BLK, EMBED), lambda i: (i, 0)),
        pl.BlockSpec((EMBED, NCODE), lambda i: (0, 0)),
        pl.BlockSpec((BLK, 1), lambda i: (i, 0)),
    ],
    out_specs=[
        pl.BlockSpec((1, 1, BLK), lambda i: (i, 0, 0)),
        pl.BlockSpec((1, 1), lambda i: (0, 0)),
    ],
    out_shape=[
        jax.ShapeDtypeStruct((NBLK, 1, BLK), jnp.int32),
        jax.ShapeDtypeStruct((1, 1), jnp.float32),
    ],
)


@functools.cache
def _make_sc_gather():
    info = plsc.get_sparse_core_info()
    ncores = info.num_cores
    nw = ncores * info.num_subcores
    bpw = NTOK // nw

    @functools.partial(
        pl.kernel,
        mesh=plsc.VectorSubcoreMesh(core_axis_name="c", subcore_axis_name="s"),
        compiler_params=pltpu.CompilerParams(use_tc_tiling_on_sc=False),
        out_type=jax.ShapeDtypeStruct((NTOK, EMBED), jnp.float32),
        scratch_types=[
            pltpu.VMEM((bpw,), jnp.int32),
            pltpu.VMEM((bpw, EMBED), jnp.float32),
            pltpu.SemaphoreType.DMA,
        ],
    )
    def sc_gather(table_hbm, idx_hbm, out_hbm, idx_v, rows_v, sem):
        wid = lax.axis_index("s") * ncores + lax.axis_index("c")
        base = wid * bpw
        pltpu.sync_copy(idx_hbm.at[pl.ds(base, bpw)], idx_v)
        pltpu.async_copy(table_hbm.at[idx_v], rows_v, sem).wait()
        pltpu.sync_copy(rows_v, out_hbm.at[pl.ds(base, bpw)])

    return sc_gather


def kernel(z, W):
    z_flat = z.reshape(NTOK, EMBED)
    z2 = jnp.sum(z_flat ** 2, axis=1, keepdims=True)
    idx3, loss = _dist_call(z_flat, (2.0 * W).T, z2)
    idx_flat = idx3.reshape(NTOK)
    z_q = _make_sc_gather()(W, idx_flat).reshape(z.shape)
    encoding_indices = idx_flat.reshape(z.shape[:-1])
    return (z_q, loss[0, 0], encoding_indices)
